# baseline (speedup 1.0000x reference)
import functools

import jax
import jax.numpy as jnp
from jax import lax
from jax.experimental import pallas as pl
from jax.experimental.pallas import tpu as pltpu
from jax.experimental.pallas import tpu_sc as plsc

B = 16
S = 512
NLM = 5000
NPAIRS = B * NLM
NTILES = 32
LANES = 16
PAIRS = NPAIRS // NTILES
WIN = 2504
PAD = 2560
NCHUNK = 2
CVEC = PAD // (NCHUNK * LANES)
CPAIR = CVEC * LANES
TOTAL = NPAIRS * 2
PLANE = S * S


def _sc_body(flow_hbm, x_hbm, y_hbm, sx_hbm, sy_hbm, g0_hbm, g1_hbm, out_hbm,
             x_v, y_v, sx_v, sy_v, g0_v, g1_v, idx_v, pts_v, row_v,
             sem_xy, sem_rest, sem_g):
    cid = lax.axis_index("c")
    sid = lax.axis_index("s")
    wid = cid * 16 + sid
    batch = wid // 2
    half = wid % 2
    w0 = half * (NLM - WIN)
    lo = half * 4
    base = batch * NLM + w0
    plane0 = batch * (2 * PLANE)

    cp_x = pltpu.async_copy(x_hbm.at[pl.ds(base, WIN)],
                            x_v.at[pl.ds(0, WIN)], sem_xy)
    cp_y = pltpu.async_copy(y_hbm.at[pl.ds(base, WIN)],
                            y_v.at[pl.ds(0, WIN)], sem_xy)
    rest = [
        pltpu.async_copy(src.at[pl.ds(base, WIN)],
                         buf.at[pl.ds(0, WIN)], sem_rest)
        for src, buf in ((sx_hbm, sx_v), (sy_hbm, sy_v),
                         (g0_hbm, g0_v), (g1_hbm, g1_v))
    ]
    cp_x.wait()
    cp_y.wait()

    lanes = lax.iota(jnp.int32, 16)
    NVEC = PAD // LANES

    def idx_body(v, _):
        p = v * LANES + lanes
        x = x_v[pl.ds(v * LANES, LANES)]
        y = y_v[pl.ds(v * LANES, LANES)]
        within = (((y >> 3) * 4 + (x >> 7)) << 10) + ((y & 7) << 7) + (x & 127)
        idx0 = jnp.where(p < WIN, plane0 + within, 0)
        idx_v[pl.ds(v * LANES, LANES)] = idx0
        idx_v[pl.ds(PAD + v * LANES, LANES)] = idx0 + PLANE
        return 0

    lax.fori_loop(0, NVEC, idx_body, 0)

    gather = pltpu.async_copy(flow_hbm.at[idx_v], pts_v, sem_g)
    for cp in rest:
        cp.wait()
    gather.wait()

    inv_half_s = jnp.float32(2.0 / S)
    scale = jnp.float32(1.0 / TOTAL)

    def acc_body(v, a):
        p = v * LANES + lanes
        off = v * LANES
        g0 = g0_v[pl.ds(off, LANES)]
        g1 = g1_v[pl.ds(off, LANES)]
        s0 = sx_v[pl.ds(off, LANES)]
        s1 = sy_v[pl.ds(off, LANES)]
        gt0 = s0.astype(jnp.float32) * inv_half_s - 1.0
        gt1 = s1.astype(jnp.float32) * inv_half_s - 1.0
        pt0 = pts_v[pl.ds(off, LANES)]
        pt1 = pts_v[pl.ds(PAD + off, LANES)]
        d0 = pt0 * g0 - gt0 * g0
        d1 = pt1 * g1 - gt1 * g1
        sq = d0 * d0 + d1 * d1
        valid = (p >= lo) & (p < lo + PAIRS)
        return a + jnp.where(valid, sq, jnp.float32(0.0))

    acc = lax.fori_loop(0, NVEC, acc_body, jnp.zeros((16,), jnp.float32))

    row_v[...] = acc * scale
    pltpu.sync_copy(row_v, out_hbm.at[wid])


@jax.jit
def _landmark_loss(flow_flat, x_f, y_f, sx_f, sy_f, g0_f, g1_f):
    mesh = plsc.VectorSubcoreMesh(core_axis_name="c", subcore_axis_name="s")
    run = functools.partial(
        pl.kernel,
        out_type=jax.ShapeDtypeStruct((NTILES, 16), jnp.float32),
        mesh=mesh,
        scratch_types=[
            pltpu.VMEM((PAD,), jnp.int32),
            pltpu.VMEM((PAD,), jnp.int32),
            pltpu.VMEM((PAD,), jnp.int32),
            pltpu.VMEM((PAD,), jnp.int32),
            pltpu.VMEM((PAD,), jnp.float32),
            pltpu.VMEM((PAD,), jnp.float32),
            pltpu.VMEM((2 * PAD,), jnp.int32),
            pltpu.VMEM((2 * PAD,), jnp.float32),
            pltpu.VMEM((16,), jnp.float32),
            pltpu.SemaphoreType.DMA,
            pltpu.SemaphoreType.DMA,
            pltpu.SemaphoreType.DMA,
        ],
        compiler_params=pltpu.CompilerParams(needs_layout_passes=False),
    )(_sc_body)
    partials = run(flow_flat, x_f, y_f, sx_f, sy_f, g0_f, g1_f)
    return jnp.sum(partials)


def kernel(flow, lm_S, lm_F, gate):
    flow_t = (
        flow.reshape(B, 2, S // 8, 8, S // 128, 128)
        .transpose(0, 1, 2, 4, 3, 5)
        .reshape(-1)
    )
    return _landmark_loss(
        flow_t,
        lm_F[:, :, 0].reshape(-1),
        lm_F[:, :, 1].reshape(-1),
        lm_S[:, :, 0].reshape(-1),
        lm_S[:, :, 1].reshape(-1),
        gate[:, :, 0].reshape(-1),
        gate[:, :, 1].reshape(-1),
    )


# speedup vs baseline: 1.2162x; 1.2162x over previous
import functools

import jax
import jax.numpy as jnp
from jax import lax
from jax.experimental import pallas as pl
from jax.experimental.pallas import tpu as pltpu
from jax.experimental.pallas import tpu_sc as plsc

B = 16
S = 512
NLM = 5000
NPAIRS = B * NLM
NTILES = 32
LANES = 16
PAIRS = NPAIRS // NTILES
WIN = 2504
PAD = 2512
NVEC = PAD // LANES
TOTAL = NPAIRS * 2
PLANE = S * S


def _sc_body(flow_hbm, x_hbm, y_hbm, sx_hbm, sy_hbm, g0_hbm, g1_hbm, out_hbm,
             x_v, y_v, sx_v, sy_v, g0_v, g1_v, idx_v, pts_v, row_v, sem):
    cid = lax.axis_index("c")
    sid = lax.axis_index("s")
    wid = cid * 16 + sid
    batch = wid // 2
    half = wid % 2
    w0 = half * (NLM - WIN)
    lo = half * 4
    plane0 = batch * (2 * PLANE)

    for src, buf in zip((x_hbm, y_hbm, sx_hbm, sy_hbm, g0_hbm, g1_hbm),
                        (x_v, y_v, sx_v, sy_v, g0_v, g1_v)):
        pltpu.sync_copy(src.at[pl.ds(batch * NLM + w0, WIN)],
                        buf.at[pl.ds(0, WIN)])

    lanes = lax.iota(jnp.int32, 16)

    def idx_body(v, _):
        p = v * LANES + lanes
        x = x_v[pl.ds(v * LANES, LANES)]
        y = y_v[pl.ds(v * LANES, LANES)]
        within = (((y >> 3) * 4 + (x >> 7)) << 10) + ((y & 7) << 7) + (x & 127)
        idx0 = jnp.where(p < WIN, plane0 + within, 0)
        idx_v[pl.ds(v * LANES, LANES)] = idx0
        idx_v[pl.ds(PAD + v * LANES, LANES)] = idx0 + PLANE
        return 0

    lax.fori_loop(0, NVEC, idx_body, 0)

    pltpu.async_copy(flow_hbm.at[idx_v], pts_v, sem).wait()

    inv_half_s = jnp.float32(2.0 / S)
    scale = jnp.float32(1.0 / TOTAL)

    def acc_body(v, acc):
        p = v * LANES + lanes
        off = v * LANES
        g0 = plsc.bitcast(g0_v[pl.ds(off, LANES)], jnp.float32)
        g1 = plsc.bitcast(g1_v[pl.ds(off, LANES)], jnp.float32)
        s0 = sx_v[pl.ds(off, LANES)]
        s1 = sy_v[pl.ds(off, LANES)]
        gt0 = s0.astype(jnp.float32) * inv_half_s - 1.0
        gt1 = s1.astype(jnp.float32) * inv_half_s - 1.0
        pt0 = pts_v[pl.ds(off, LANES)]
        pt1 = pts_v[pl.ds(PAD + off, LANES)]
        d0 = pt0 * g0 - gt0 * g0
        d1 = pt1 * g1 - gt1 * g1
        sq = d0 * d0 + d1 * d1
        valid = (p >= lo) & (p < lo + PAIRS)
        return acc + jnp.where(valid, sq, jnp.float32(0.0))

    acc = lax.fori_loop(0, NVEC, acc_body, jnp.zeros((16,), jnp.float32))
    row_v[...] = acc * scale
    pltpu.sync_copy(row_v, out_hbm.at[wid])


@jax.jit
def _landmark_loss(flow_flat, x_f, y_f, sx_f, sy_f, g0_f, g1_f):
    mesh = plsc.VectorSubcoreMesh(core_axis_name="c", subcore_axis_name="s")
    run = functools.partial(
        pl.kernel,
        out_type=jax.ShapeDtypeStruct((NTILES, 16), jnp.float32),
        mesh=mesh,
        scratch_types=[
            pltpu.VMEM((PAD,), jnp.int32),
            pltpu.VMEM((PAD,), jnp.int32),
            pltpu.VMEM((PAD,), jnp.int32),
            pltpu.VMEM((PAD,), jnp.int32),
            pltpu.VMEM((PAD,), jnp.int32),
            pltpu.VMEM((PAD,), jnp.int32),
            pltpu.VMEM((2 * PAD,), jnp.int32),
            pltpu.VMEM((2 * PAD,), jnp.float32),
            pltpu.VMEM((16,), jnp.float32),
            pltpu.SemaphoreType.DMA,
        ],
        compiler_params=pltpu.CompilerParams(needs_layout_passes=False),
    )(_sc_body)
    partials = run(flow_flat, x_f, y_f, sx_f, sy_f, g0_f, g1_f)
    return jnp.sum(partials)


def kernel(flow, lm_S, lm_F, gate):
    flow_t = (
        flow.reshape(B, 2, S // 8, 8, S // 128, 128)
        .transpose(0, 1, 2, 4, 3, 5)
        .reshape(-1)
    )
    gate_i = jax.lax.bitcast_convert_type(gate, jnp.int32)
    return _landmark_loss(
        flow_t,
        lm_F[:, :, 0].reshape(-1),
        lm_F[:, :, 1].reshape(-1),
        lm_S[:, :, 0].reshape(-1),
        lm_S[:, :, 1].reshape(-1),
        gate_i[:, :, 0].reshape(-1),
        gate_i[:, :, 1].reshape(-1),
    )
